# baseline (speedup 1.0000x reference)
import functools

import jax
import jax.numpy as jnp
from jax import lax
from jax.experimental import pallas as pl
from jax.experimental.pallas import tpu as pltpu
from jax.experimental.pallas import tpu_sc as plsc

_N = 1048576
_F = 1024
_TABLE = 65536
_NC = 2
_NS = 16
_NW = _NC * _NS
_EPW = _N // _NW
_CHUNK = 16384
_NCHUNK = _EPW // _CHUNK
_GSIZE = 128
_KFIRE = 16
_NGROUP = _CHUNK // (_GSIZE * _KFIRE)

_P1 = 1
_P2 = -1640531535
_P3 = 805459861

_mesh = plsc.VectorSubcoreMesh(core_axis_name="c", subcore_axis_name="s")


@functools.partial(
    pl.kernel,
    out_type=jax.ShapeDtypeStruct((_N,), jnp.float32),
    mesh=_mesh,
    scratch_types=[
        pltpu.VMEM((_CHUNK,), jnp.int32),
        pltpu.VMEM((_CHUNK,), jnp.int32),
        pltpu.VMEM((_CHUNK,), jnp.int32),
        pltpu.VMEM((_CHUNK,), jnp.int32),
        pltpu.VMEM((_CHUNK,), jnp.int32),
        pltpu.VMEM((_CHUNK,), jnp.float32),
        pltpu.SemaphoreType.DMA,
    ],
)
def _hash_gather(index_hbm, data_hbm, out_hbm, f_v, x_v, y_v, z_v, idx_v,
                 val_v, sem):
    wid = lax.axis_index("s") * _NC + lax.axis_index("c")
    base = wid * _EPW

    for ci in range(_NCHUNK):
        cb = base + ci * _CHUNK
        pltpu.sync_copy(index_hbm.at[0, pl.ds(cb, _CHUNK)], f_v)
        pltpu.sync_copy(index_hbm.at[1, pl.ds(cb, _CHUNK)], x_v)
        pltpu.sync_copy(index_hbm.at[2, pl.ds(cb, _CHUNK)], y_v)
        pltpu.sync_copy(index_hbm.at[3, pl.ds(cb, _CHUNK)], z_v)

        def hash_body(i, carry):
            s = pl.ds(i * 16, 16)
            h = (x_v[s] ^ jnp.int32(_P1)) ^ (y_v[s] * jnp.int32(_P2)) \
                ^ (z_v[s] * jnp.int32(_P3))
            idx_v[s] = (f_v[s] << 16) | (h & jnp.int32(0xFFFF))
            return carry

        lax.fori_loop(0, _CHUNK // 16, hash_body, 0)

        def gather_group(g, carry):
            gb = g * (_GSIZE * _KFIRE)
            copies = [
                pltpu.async_copy(
                    data_hbm.at[idx_v.at[pl.ds(gb + j * _GSIZE, _GSIZE)]],
                    val_v.at[pl.ds(gb + j * _GSIZE, _GSIZE)],
                    sem,
                )
                for j in range(_KFIRE)
            ]
            for cp in copies:
                cp.wait()
            return carry

        lax.fori_loop(0, _NGROUP, gather_group, 0)

        pltpu.sync_copy(val_v, out_hbm.at[pl.ds(cb, _CHUNK)])


def kernel(index, data):
    return _hash_gather(index, data.reshape(-1))


# speedup vs baseline: 3.1011x; 3.1011x over previous
import functools

import jax
import jax.numpy as jnp
from jax import lax
from jax.experimental import pallas as pl
from jax.experimental.pallas import tpu as pltpu
from jax.experimental.pallas import tpu_sc as plsc

_N = 1048576
_F = 1024
_TABLE = 65536
_NC = 2
_NS = 16
_NW = _NC * _NS
_EPW = _N // _NW
_CHUNK = 16384
_NCHUNK = _EPW // _CHUNK
_GSIZE = 128
_KFIRE = 16
_NGROUP = _CHUNK // (_GSIZE * _KFIRE)

_P1 = 1
_P2 = -1640531535
_P3 = 805459861

_mesh = plsc.VectorSubcoreMesh(core_axis_name="c", subcore_axis_name="s")


@functools.partial(
    pl.kernel,
    out_type=jax.ShapeDtypeStruct((_N,), jnp.float32),
    mesh=_mesh,
    scratch_types=[
        pltpu.VMEM((_CHUNK,), jnp.int32),
        pltpu.VMEM((_CHUNK,), jnp.int32),
        pltpu.VMEM((_CHUNK,), jnp.int32),
        pltpu.VMEM((_CHUNK,), jnp.int32),
        pltpu.VMEM((_CHUNK,), jnp.int32),
        pltpu.VMEM((_CHUNK,), jnp.float32),
        pltpu.SemaphoreType.DMA,
    ],
)
def _hash_gather(index_hbm, data_hbm, out_hbm, f_v, x_v, y_v, z_v, idx_v,
                 val_v, sem):
    wid = lax.axis_index("s") * _NC + lax.axis_index("c")
    base = wid * _EPW

    for ci in range(_NCHUNK):
        cb = base + ci * _CHUNK
        pltpu.sync_copy(index_hbm.at[0, pl.ds(cb, _CHUNK)], f_v)
        pltpu.sync_copy(index_hbm.at[1, pl.ds(cb, _CHUNK)], x_v)
        pltpu.sync_copy(index_hbm.at[2, pl.ds(cb, _CHUNK)], y_v)
        pltpu.sync_copy(index_hbm.at[3, pl.ds(cb, _CHUNK)], z_v)

        def hash_body(i, carry):
            s = pl.ds(i * 16, 16)
            h = (x_v[s] ^ jnp.int32(_P1)) ^ (y_v[s] * jnp.int32(_P2)) \
                ^ (z_v[s] * jnp.int32(_P3))
            slot = h & jnp.int32(0xFFFF)
            f = f_v[s]
            idx_v[s] = (
                ((f >> 3) << 19)
                | ((slot >> 7) << 10)
                | ((f & jnp.int32(7)) << 7)
                | (slot & jnp.int32(127))
            )
            return carry

        lax.fori_loop(0, _CHUNK // 16, hash_body, 0)

        def gather_group(g, carry):
            gb = g * (_GSIZE * _KFIRE)
            copies = [
                pltpu.async_copy(
                    data_hbm.at[idx_v.at[pl.ds(gb + j * _GSIZE, _GSIZE)]],
                    val_v.at[pl.ds(gb + j * _GSIZE, _GSIZE)],
                    sem,
                )
                for j in range(_KFIRE)
            ]
            for cp in copies:
                cp.wait()
            return carry

        lax.fori_loop(0, _NGROUP, gather_group, 0)

        pltpu.sync_copy(val_v, out_hbm.at[pl.ds(cb, _CHUNK)])


def kernel(index, data):
    data_flat = data.reshape(128, 8, 512, 128).transpose(0, 2, 1, 3).reshape(-1)
    return _hash_gather(index, data_flat)


# speedup vs baseline: 3.9050x; 1.2592x over previous
import functools

import jax
import jax.numpy as jnp
from jax import lax
from jax.experimental import pallas as pl
from jax.experimental.pallas import tpu as pltpu
from jax.experimental.pallas import tpu_sc as plsc

_N = 1048576
_F = 1024
_TABLE = 65536
_NC = 2
_NS = 16
_NW = _NC * _NS
_EPW = _N // _NW
_CHUNK = 16384
_NCHUNK = _EPW // _CHUNK
_GSIZE = 128
_KFIRE = 16
_NGROUP = _CHUNK // (_GSIZE * _KFIRE)

_P1 = 1
_P2 = -1640531535
_P3 = 805459861

_mesh = plsc.VectorSubcoreMesh(core_axis_name="c", subcore_axis_name="s")


@functools.partial(
    pl.kernel,
    out_type=jax.ShapeDtypeStruct((_N,), jnp.float32),
    mesh=_mesh,
    scratch_types=[
        pltpu.VMEM((_CHUNK,), jnp.int32),
        pltpu.VMEM((_CHUNK,), jnp.int32),
        pltpu.VMEM((_CHUNK,), jnp.int32),
        pltpu.VMEM((_CHUNK,), jnp.int32),
        pltpu.VMEM((_CHUNK,), jnp.int32),
        pltpu.VMEM((_CHUNK,), jnp.float32),
        pltpu.SemaphoreType.DMA,
    ],
)
def _hash_gather(index_hbm, data_hbm, out_hbm, f_v, x_v, y_v, z_v, idx_v,
                 val_v, sem):
    wid = lax.axis_index("s") * _NC + lax.axis_index("c")
    base = wid * _EPW

    for ci in range(_NCHUNK):
        cb = base + ci * _CHUNK
        pltpu.sync_copy(index_hbm.at[0, pl.ds(cb, _CHUNK)], f_v)
        pltpu.sync_copy(index_hbm.at[1, pl.ds(cb, _CHUNK)], x_v)
        pltpu.sync_copy(index_hbm.at[2, pl.ds(cb, _CHUNK)], y_v)
        pltpu.sync_copy(index_hbm.at[3, pl.ds(cb, _CHUNK)], z_v)

        def fire_body(g, carry):
            for u in range(_GSIZE // 16):
                s = pl.ds(g * _GSIZE + u * 16, 16)
                h = (x_v[s] ^ jnp.int32(_P1)) ^ (y_v[s] * jnp.int32(_P2)) \
                    ^ (z_v[s] * jnp.int32(_P3))
                slot = h & jnp.int32(0xFFFF)
                f = f_v[s]
                idx_v[s] = (
                    ((f >> 3) << 19)
                    | ((slot >> 7) << 10)
                    | ((f & jnp.int32(7)) << 7)
                    | (slot & jnp.int32(127))
                )
            gb = g * _GSIZE
            pltpu.async_copy(
                data_hbm.at[idx_v.at[pl.ds(gb, _GSIZE)]],
                val_v.at[pl.ds(gb, _GSIZE)],
                sem,
            )
            return carry

        lax.fori_loop(0, _CHUNK // _GSIZE, fire_body, 0)
        pltpu.make_async_copy(
            data_hbm.at[pl.ds(0, _CHUNK)], val_v, sem
        ).wait()

        pltpu.sync_copy(val_v, out_hbm.at[pl.ds(cb, _CHUNK)])


def kernel(index, data):
    data_flat = data.reshape(128, 8, 512, 128).transpose(0, 2, 1, 3).reshape(-1)
    return _hash_gather(index, data_flat)


# speedup vs baseline: 4.1216x; 1.0555x over previous
import functools

import jax
import jax.numpy as jnp
from jax import lax
from jax.experimental import pallas as pl
from jax.experimental.pallas import tpu as pltpu
from jax.experimental.pallas import tpu_sc as plsc

_N = 1048576
_NC = 2
_NS = 16
_NW = _NC * _NS
_EPW = _N // _NW
_CHUNK = 8192
_NCHUNK = _EPW // _CHUNK
_GSIZE = 128

_P1 = 1
_P2 = -1640531535
_P3 = 805459861

_mesh = plsc.VectorSubcoreMesh(core_axis_name="c", subcore_axis_name="s")

_in_buf = lambda: pltpu.VMEM((_CHUNK,), jnp.int32)


@functools.partial(
    pl.kernel,
    out_type=jax.ShapeDtypeStruct((_N,), jnp.float32),
    mesh=_mesh,
    scratch_types=[
        [_in_buf() for _ in range(4)],
        [_in_buf() for _ in range(4)],
        [pltpu.VMEM((_CHUNK,), jnp.int32) for _ in range(2)],
        [pltpu.VMEM((_CHUNK,), jnp.float32) for _ in range(2)],
        pltpu.SemaphoreType.DMA,
        [pltpu.SemaphoreType.DMA for _ in range(2)],
        [pltpu.SemaphoreType.DMA for _ in range(2)],
    ],
)
def _hash_gather(index_hbm, data_hbm, out_hbm, in0, in1, idxs, vals,
                 sem_g, sem_in, sem_out):
    wid = lax.axis_index("s") * _NC + lax.axis_index("c")
    base = wid * _EPW
    insets = (in0, in1)

    def stage(ci):
        cb = base + ci * _CHUNK
        bufs = insets[ci % 2]
        for r in range(4):
            pltpu.async_copy(index_hbm.at[r, pl.ds(cb, _CHUNK)], bufs[r],
                             sem_in[ci % 2])

    def wait_stage(ci):
        cb = base + ci * _CHUNK
        bufs = insets[ci % 2]
        for r in range(4):
            pltpu.make_async_copy(index_hbm.at[r, pl.ds(cb, _CHUNK)],
                                  bufs[r], sem_in[ci % 2]).wait()

    stage(0)
    for ci in range(_NCHUNK):
        b = ci % 2
        f_v, x_v, y_v, z_v = insets[b]
        idx_v, val_v = idxs[b], vals[b]
        cb = base + ci * _CHUNK

        if ci + 1 < _NCHUNK:
            stage(ci + 1)
        wait_stage(ci)
        if ci >= 2:
            pltpu.make_async_copy(
                val_v, out_hbm.at[pl.ds(cb - 2 * _CHUNK, _CHUNK)], sem_out[b]
            ).wait()

        def fire_body(g, carry):
            for u in range(_GSIZE // 16):
                s = pl.ds(g * _GSIZE + u * 16, 16)
                h = (x_v[s] ^ jnp.int32(_P1)) ^ (y_v[s] * jnp.int32(_P2)) \
                    ^ (z_v[s] * jnp.int32(_P3))
                slot = h & jnp.int32(0xFFFF)
                f = f_v[s]
                idx_v[s] = (
                    ((f >> 3) << 19)
                    | ((slot >> 7) << 10)
                    | ((f & jnp.int32(7)) << 7)
                    | (slot & jnp.int32(127))
                )
            gb = g * _GSIZE
            pltpu.async_copy(
                data_hbm.at[idx_v.at[pl.ds(gb, _GSIZE)]],
                val_v.at[pl.ds(gb, _GSIZE)],
                sem_g,
            )
            return carry

        lax.fori_loop(0, _CHUNK // _GSIZE, fire_body, 0)
        pltpu.make_async_copy(
            data_hbm.at[pl.ds(0, _CHUNK)], val_v, sem_g
        ).wait()

        pltpu.async_copy(val_v, out_hbm.at[pl.ds(cb, _CHUNK)], sem_out[b])

    for ci in range(_NCHUNK - 2, _NCHUNK):
        b = ci % 2
        cb = base + ci * _CHUNK
        pltpu.make_async_copy(
            vals[b], out_hbm.at[pl.ds(cb, _CHUNK)], sem_out[b]
        ).wait()


def kernel(index, data):
    data_flat = data.reshape(128, 8, 512, 128).transpose(0, 2, 1, 3).reshape(-1)
    return _hash_gather(index, data_flat)


# speedup vs baseline: 4.2192x; 1.0237x over previous
import functools

import jax
import jax.numpy as jnp
from jax import lax
from jax.experimental import pallas as pl
from jax.experimental.pallas import tpu as pltpu
from jax.experimental.pallas import tpu_sc as plsc

_N = 1048576
_NC = 2
_NS = 16
_NW = _NC * _NS
_EPW = _N // _NW
_CHUNK = 8192
_NCHUNK = _EPW // _CHUNK
_GSIZE = 128

_P1 = 1
_P2 = -1640531535
_P3 = 805459861

_mesh = plsc.VectorSubcoreMesh(core_axis_name="c", subcore_axis_name="s")

_in_buf = lambda: pltpu.VMEM((_CHUNK,), jnp.int32)


@functools.partial(
    pl.kernel,
    out_type=jax.ShapeDtypeStruct((_N,), jnp.float32),
    mesh=_mesh,
    scratch_types=[
        [_in_buf() for _ in range(4)],
        [_in_buf() for _ in range(4)],
        [pltpu.VMEM((_CHUNK,), jnp.int32) for _ in range(2)],
        [pltpu.VMEM((_CHUNK,), jnp.float32) for _ in range(2)],
        [pltpu.SemaphoreType.DMA for _ in range(2)],
        [pltpu.SemaphoreType.DMA for _ in range(2)],
        [pltpu.SemaphoreType.DMA for _ in range(2)],
    ],
)
def _hash_gather(index_hbm, data_hbm, out_hbm, in0, in1, idxs, vals,
                 sem_g, sem_in, sem_out):
    wid = lax.axis_index("s") * _NC + lax.axis_index("c")
    base = wid * _EPW
    insets = (in0, in1)

    def stage(ci):
        cb = base + ci * _CHUNK
        bufs = insets[ci % 2]
        for r in range(4):
            pltpu.async_copy(index_hbm.at[r, pl.ds(cb, _CHUNK)], bufs[r],
                             sem_in[ci % 2])

    def wait_stage(ci):
        cb = base + ci * _CHUNK
        bufs = insets[ci % 2]
        for r in range(4):
            pltpu.make_async_copy(index_hbm.at[r, pl.ds(cb, _CHUNK)],
                                  bufs[r], sem_in[ci % 2]).wait()

    def drain_and_flush(ci):
        b = ci % 2
        cb = base + ci * _CHUNK
        pltpu.make_async_copy(
            data_hbm.at[pl.ds(0, _CHUNK)], vals[b], sem_g[b]
        ).wait()
        pltpu.async_copy(vals[b], out_hbm.at[pl.ds(cb, _CHUNK)], sem_out[b])

    stage(0)
    for ci in range(_NCHUNK):
        b = ci % 2
        f_v, x_v, y_v, z_v = insets[b]
        idx_v, val_v = idxs[b], vals[b]
        cb = base + ci * _CHUNK

        if ci + 1 < _NCHUNK:
            stage(ci + 1)
        wait_stage(ci)
        if ci >= 2:
            pltpu.make_async_copy(
                val_v, out_hbm.at[pl.ds(cb - 2 * _CHUNK, _CHUNK)], sem_out[b]
            ).wait()

        def fire_body(g, carry):
            for u in range(_GSIZE // 16):
                s = pl.ds(g * _GSIZE + u * 16, 16)
                h = (x_v[s] ^ jnp.int32(_P1)) ^ (y_v[s] * jnp.int32(_P2)) \
                    ^ (z_v[s] * jnp.int32(_P3))
                slot = h & jnp.int32(0xFFFF)
                f = f_v[s]
                idx_v[s] = (
                    ((f >> 3) << 19)
                    | ((slot >> 7) << 10)
                    | ((f & jnp.int32(7)) << 7)
                    | (slot & jnp.int32(127))
                )
            gb = g * _GSIZE
            pltpu.async_copy(
                data_hbm.at[idx_v.at[pl.ds(gb, _GSIZE)]],
                val_v.at[pl.ds(gb, _GSIZE)],
                sem_g[b],
            )
            return carry

        lax.fori_loop(0, _CHUNK // _GSIZE, fire_body, 0)
        if ci >= 1:
            drain_and_flush(ci - 1)

    drain_and_flush(_NCHUNK - 1)
    for ci in range(_NCHUNK - 2, _NCHUNK):
        b = ci % 2
        cb = base + ci * _CHUNK
        pltpu.make_async_copy(
            vals[b], out_hbm.at[pl.ds(cb, _CHUNK)], sem_out[b]
        ).wait()


def kernel(index, data):
    data_flat = data.reshape(128, 8, 512, 128).transpose(0, 2, 1, 3).reshape(-1)
    return _hash_gather(index, data_flat)
